# Initial kernel scaffold; baseline (speedup 1.0000x reference)
import jax
import jax.numpy as jnp
from jax.experimental import pallas as pl


def kernel(x, frame_indices, pe):
    raise NotImplementedError("write your pallas kernel here")


# speedup vs baseline: 1.8431x; 1.8431x over previous
import functools

import jax
import jax.numpy as jnp
from jax import lax
from jax.experimental import pallas as pl
from jax.experimental.pallas import tpu as pltpu
from jax.experimental.pallas import tpu_sc as plsc

_INFO = plsc.get_sparse_core_info()
_NC, _NS, _L = _INFO.num_cores, _INFO.num_subcores, _INFO.num_lanes
_NW = _NC * _NS

_D = 128
_N = 1024 * 200
_PER_W = _N // _NW
_C = 128
_NCHUNK = _PER_W // _C


def _body(x_hbm, idx_hbm, pe_hbm, out_hbm, idx_v, xa_v, pea_v, sem):
    wid = lax.axis_index("s") * _NC + lax.axis_index("c")
    base = wid * _PER_W

    def chunk(g, carry):
        b = base + g * _C
        pltpu.sync_copy(idx_hbm.at[pl.ds(b, _C)], idx_v)
        gather = pltpu.async_copy(pe_hbm.at[idx_v], pea_v, sem)
        pltpu.sync_copy(x_hbm.at[pl.ds(b, _C)], xa_v)
        gather.wait()

        def row(r, c2):
            for d in range(_D // _L):
                s = pl.ds(d * _L, _L)
                xa_v[r, s] = xa_v[r, s] + pea_v[r, s]
            return c2

        lax.fori_loop(0, _C, row, 0, unroll=2)
        pltpu.sync_copy(xa_v, out_hbm.at[pl.ds(b, _C)])
        return carry

    lax.fori_loop(0, _NCHUNK, chunk, 0)


@jax.jit
def _run(x2, idx32, pe):
    mesh = plsc.VectorSubcoreMesh(core_axis_name="c", subcore_axis_name="s")
    kfn = pl.kernel(
        _body,
        out_type=jax.ShapeDtypeStruct((_N, _D), jnp.float32),
        mesh=mesh,
        scratch_types=[
            pltpu.VMEM((_C,), jnp.int32),
            pltpu.VMEM((_C, _D), jnp.float32),
            pltpu.VMEM((_C, _D), jnp.float32),
            pltpu.SemaphoreType.DMA,
        ],
    )
    return kfn(x2, idx32, pe)


def kernel(x, frame_indices, pe):
    B, T, D = x.shape
    x2 = x.reshape(B * T, D)
    idx32 = frame_indices.reshape(-1).astype(jnp.int32)
    out = _run(x2, idx32, pe)
    return out.reshape(B, T, D)


# speedup vs baseline: 2.6562x; 1.4411x over previous
import jax
import jax.numpy as jnp
from jax import lax
from jax.experimental import pallas as pl
from jax.experimental.pallas import tpu as pltpu
from jax.experimental.pallas import tpu_sc as plsc

_INFO = plsc.get_sparse_core_info()
_NC, _NS, _L = _INFO.num_cores, _INFO.num_subcores, _INFO.num_lanes
_NW = _NC * _NS

_D = 128
_N = 1024 * 200
_PER_W = _N // _NW
_C = 128
_NCHUNK = _PER_W // _C
_NBUF = 2


def _body(x_hbm, idx2_hbm, pe_hbm, out_hbm,
          idxw, xa0, xa1, pea0, pea1, oa0, oa1,
          sx0, sx1, sg0, sg1, so0, so1):
    xa = (xa0, xa1)
    pea = (pea0, pea1)
    oa = (oa0, oa1)
    sx = (sx0, sx1)
    sg = (sg0, sg1)
    so = (so0, so1)

    wid = lax.axis_index("s") * _NC + lax.axis_index("c")
    base = wid * _PER_W
    pltpu.sync_copy(idx2_hbm.at[wid], idxw)

    def fill(g, i):
        pltpu.async_copy(x_hbm.at[pl.ds(base + g * _C, _C)], xa[i], sx[i])
        pltpu.async_copy(pe_hbm.at[idxw.at[g]], pea[i], sg[i])

    fill(0, 0)
    fill(1, 1)

    def outer(k, carry):
        g0 = k * _NBUF
        for i in range(_NBUF):
            g = g0 + i
            pltpu.make_async_copy(x_hbm.at[pl.ds(0, _C)], xa[i], sx[i]).wait()
            pltpu.make_async_copy(pe_hbm.at[idxw.at[0]], pea[i], sg[i]).wait()

            @pl.when(g0 >= _NBUF)
            def _():
                pltpu.make_async_copy(
                    oa[i], out_hbm.at[pl.ds(0, _C)], so[i]).wait()

            def row(r, c2):
                for d in range(_D // _L):
                    s = pl.ds(d * _L, _L)
                    oa[i][r, s] = xa[i][r, s] + pea[i][r, s]
                return c2

            lax.fori_loop(0, _C, row, 0, unroll=2)

            @pl.when(g + _NBUF < _NCHUNK)
            def _():
                fill(g + _NBUF, i)

            pltpu.async_copy(oa[i], out_hbm.at[pl.ds(base + g * _C, _C)], so[i])
        return carry

    lax.fori_loop(0, _NCHUNK // _NBUF, outer, 0)
    for i in range(_NBUF):
        pltpu.make_async_copy(oa[i], out_hbm.at[pl.ds(0, _C)], so[i]).wait()


@jax.jit
def _run(x2, idx2, pe):
    mesh = plsc.VectorSubcoreMesh(core_axis_name="c", subcore_axis_name="s")
    kfn = pl.kernel(
        _body,
        out_type=jax.ShapeDtypeStruct((_N, _D), jnp.float32),
        mesh=mesh,
        scratch_types=[
            pltpu.VMEM((_NCHUNK, _C), jnp.int32),
            pltpu.VMEM((_C, _D), jnp.float32),
            pltpu.VMEM((_C, _D), jnp.float32),
            pltpu.VMEM((_C, _D), jnp.float32),
            pltpu.VMEM((_C, _D), jnp.float32),
            pltpu.VMEM((_C, _D), jnp.float32),
            pltpu.VMEM((_C, _D), jnp.float32),
            pltpu.SemaphoreType.DMA,
            pltpu.SemaphoreType.DMA,
            pltpu.SemaphoreType.DMA,
            pltpu.SemaphoreType.DMA,
            pltpu.SemaphoreType.DMA,
            pltpu.SemaphoreType.DMA,
        ],
    )
    return kfn(x2, idx2, pe)


def kernel(x, frame_indices, pe):
    B, T, D = x.shape
    x2 = x.reshape(B * T, D)
    idx2 = frame_indices.reshape(_NW, _NCHUNK, _C).astype(jnp.int32)
    out = _run(x2, idx2, pe)
    return out.reshape(B, T, D)


# speedup vs baseline: 2.7001x; 1.0165x over previous
import jax
import jax.numpy as jnp
from jax import lax
from jax.experimental import pallas as pl
from jax.experimental.pallas import tpu as pltpu
from jax.experimental.pallas import tpu_sc as plsc

_INFO = plsc.get_sparse_core_info()
_NC, _NS, _L = _INFO.num_cores, _INFO.num_subcores, _INFO.num_lanes
_NW = _NC * _NS

_D = 128
_N = 1024 * 200
_PER_W = _N // _NW
_C = 128
_NCHUNK = _PER_W // _C
_NBUF = 2


def _body(x_hbm, idx2_hbm, pe_hbm, out_hbm,
          idxw, pe_sh, xa0, xa1, pea0, pea1, oa0, oa1,
          sx0, sx1, sg0, sg1, so0, so1):
    xa = (xa0, xa1)
    pea = (pea0, pea1)
    oa = (oa0, oa1)
    sx = (sx0, sx1)
    sg = (sg0, sg1)
    so = (so0, so1)

    sid = lax.axis_index("s")
    wid = sid * _NC + lax.axis_index("c")
    base = wid * _PER_W

    @pl.when(sid == 0)
    def _():
        pltpu.sync_copy(pe_hbm, pe_sh)

    pltpu.sync_copy(idx2_hbm.at[wid], idxw)
    plsc.subcore_barrier()

    def fill(g, i):
        pltpu.async_copy(x_hbm.at[pl.ds(base + g * _C, _C)], xa[i], sx[i])
        pltpu.async_copy(pe_sh.at[idxw.at[g]], pea[i], sg[i])

    fill(0, 0)
    fill(1, 1)

    def outer(k, carry):
        g0 = k * _NBUF
        for i in range(_NBUF):
            g = g0 + i
            pltpu.make_async_copy(x_hbm.at[pl.ds(0, _C)], xa[i], sx[i]).wait()
            pltpu.make_async_copy(pe_sh.at[idxw.at[0]], pea[i], sg[i]).wait()

            @pl.when(g0 >= _NBUF)
            def _():
                pltpu.make_async_copy(
                    oa[i], out_hbm.at[pl.ds(0, _C)], so[i]).wait()

            def row(r, c2):
                for d in range(_D // _L):
                    s = pl.ds(d * _L, _L)
                    oa[i][r, s] = xa[i][r, s] + pea[i][r, s]
                return c2

            lax.fori_loop(0, _C, row, 0, unroll=2)

            @pl.when(g + _NBUF < _NCHUNK)
            def _():
                fill(g + _NBUF, i)

            pltpu.async_copy(oa[i], out_hbm.at[pl.ds(base + g * _C, _C)], so[i])
        return carry

    lax.fori_loop(0, _NCHUNK // _NBUF, outer, 0)
    for i in range(_NBUF):
        pltpu.make_async_copy(oa[i], out_hbm.at[pl.ds(0, _C)], so[i]).wait()


@jax.jit
def _run(x2, idx2, pe):
    mesh = plsc.VectorSubcoreMesh(core_axis_name="c", subcore_axis_name="s")
    kfn = pl.kernel(
        _body,
        out_type=jax.ShapeDtypeStruct((_N, _D), jnp.float32),
        mesh=mesh,
        scratch_types=[
            pltpu.VMEM((_NCHUNK, _C), jnp.int32),
            pltpu.VMEM_SHARED((256, _D), jnp.float32),
            pltpu.VMEM((_C, _D), jnp.float32),
            pltpu.VMEM((_C, _D), jnp.float32),
            pltpu.VMEM((_C, _D), jnp.float32),
            pltpu.VMEM((_C, _D), jnp.float32),
            pltpu.VMEM((_C, _D), jnp.float32),
            pltpu.VMEM((_C, _D), jnp.float32),
            pltpu.SemaphoreType.DMA,
            pltpu.SemaphoreType.DMA,
            pltpu.SemaphoreType.DMA,
            pltpu.SemaphoreType.DMA,
            pltpu.SemaphoreType.DMA,
            pltpu.SemaphoreType.DMA,
        ],
    )
    return kfn(x2, idx2, pe)


def kernel(x, frame_indices, pe):
    B, T, D = x.shape
    x2 = x.reshape(B * T, D)
    idx2 = frame_indices.reshape(_NW, _NCHUNK, _C).astype(jnp.int32)
    out = _run(x2, idx2, pe)
    return out.reshape(B, T, D)


# speedup vs baseline: 8.0334x; 2.9753x over previous
import jax
import jax.numpy as jnp
from jax import lax
from jax.experimental import pallas as pl
from jax.experimental.pallas import tpu as pltpu
from jax.experimental.pallas import tpu_sc as plsc

_INFO = plsc.get_sparse_core_info()
_NC, _NS, _L = _INFO.num_cores, _INFO.num_subcores, _INFO.num_lanes
_NW = _NC * _NS

_D = 128
_N = 1024 * 200
_PER_W = _N // _NW
_C = 128
_NCHUNK = _PER_W // _C
_NBUF = 5
_PF = 3


def _body(x_hbm, idx2_hbm, pe_hbm, out_hbm,
          idxw, pe_sh, xa0, xa1, xa2, xa3, xa4, sx, sg, so):
    xa = (xa0, xa1, xa2, xa3, xa4)
    sid = lax.axis_index("s")
    wid = sid * _NC + lax.axis_index("c")
    base = wid * _PER_W

    @pl.when(sid == 0)
    def _():
        pltpu.sync_copy(pe_hbm, pe_sh)

    pltpu.sync_copy(idx2_hbm.at[wid], idxw)
    plsc.subcore_barrier()

    def fill(g, i):
        pltpu.async_copy(x_hbm.at[pl.ds(base + g * _C, _C)], xa[i], sx.at[i])

    for g in range(_PF):
        fill(g, g)

    def rnd(r, carry):
        for i in range(_NBUF):
            g = r * _NBUF + i
            j = (i + _PF) % _NBUF

            @pl.when(g >= _NBUF - _PF)
            def _():
                pltpu.make_async_copy(
                    xa[j], out_hbm.at[pl.ds(0, _C)], so.at[j]).wait()

            @pl.when(g + _PF < _NCHUNK)
            def _():
                fill(g + _PF, j)

            pltpu.make_async_copy(
                x_hbm.at[pl.ds(0, _C)], xa[i], sx.at[i]).wait()
            pltpu.async_copy(
                pe_sh.at[idxw.at[g]], xa[i], sg.at[i], add=True)
            pltpu.make_async_copy(
                pe_sh.at[idxw.at[0]], xa[i], sg.at[i]).wait()
            pltpu.async_copy(
                xa[i], out_hbm.at[pl.ds(base + g * _C, _C)], so.at[i])
        return carry

    lax.fori_loop(0, _NCHUNK // _NBUF, rnd, 0)
    for i in range(_NBUF - _PF):
        b = _NBUF - (_NBUF - _PF) + i
        pltpu.make_async_copy(xa[b], out_hbm.at[pl.ds(0, _C)], so.at[b]).wait()


@jax.jit
def _run(x2, idx2, pe):
    mesh = plsc.VectorSubcoreMesh(core_axis_name="c", subcore_axis_name="s")
    kfn = pl.kernel(
        _body,
        out_type=jax.ShapeDtypeStruct((_N, _D), jnp.float32),
        mesh=mesh,
        scratch_types=[
            pltpu.VMEM((_NCHUNK, _C), jnp.int32),
            pltpu.VMEM_SHARED((256, _D), jnp.float32),
            pltpu.VMEM((_C, _D), jnp.float32),
            pltpu.VMEM((_C, _D), jnp.float32),
            pltpu.VMEM((_C, _D), jnp.float32),
            pltpu.VMEM((_C, _D), jnp.float32),
            pltpu.VMEM((_C, _D), jnp.float32),
            pltpu.SemaphoreType.DMA((_NBUF,)),
            pltpu.SemaphoreType.DMA((_NBUF,)),
            pltpu.SemaphoreType.DMA((_NBUF,)),
        ],
    )
    return kfn(x2, idx2, pe)


def kernel(x, frame_indices, pe):
    B, T, D = x.shape
    x2 = x.reshape(B * T, D)
    idx2 = frame_indices.reshape(_NW, _NCHUNK, _C).astype(jnp.int32)
    out = _run(x2, idx2, pe)
    return out.reshape(B, T, D)
